# baseline (speedup 1.0000x reference)
import functools
import math

import jax
import jax.numpy as jnp
from jax.experimental import pallas as pl
from jax.experimental.pallas import tpu as pltpu

EMBED = 1024
HEADS = 16
HEAD_DIM = 64
SCALE = HEAD_DIM ** -0.5
WINDOW = 2
FAR = 4
S = 2048
BQ = 256
EXTB = 256

_LOG1E4 = math.log(10000.0)


def _qkv_rope_kernel(x_ref, wq_ref, wk_ref, wv_ref, b_ref, o_ref):
    qi = pl.program_id(0)
    y = jnp.concatenate([
        jnp.dot(x_ref[...], wq_ref[...], preferred_element_type=jnp.float32),
        jnp.dot(x_ref[...], wk_ref[...], preferred_element_type=jnp.float32),
        jnp.dot(x_ref[...], wv_ref[...], preferred_element_type=jnp.float32),
    ], axis=1)
    y = y + b_ref[...]
    bq, n = y.shape
    col = jax.lax.broadcasted_iota(jnp.int32, (bq, 128), 1)
    d = col % HEAD_DIM
    dr = d % (HEAD_DIM // 2)
    hi = d >= (HEAD_DIM // 2)
    inv = jnp.exp(dr.astype(jnp.float32) * (-_LOG1E4 / (HEAD_DIM // 2)))
    row = jax.lax.broadcasted_iota(jnp.int32, (bq, 128), 0)
    pos = (qi * bq + row).astype(jnp.float32)
    ang = pos * inv
    c = jnp.cos(ang)
    sg = jnp.sin(ang)
    sg = jnp.where(hi, sg, -sg)
    ones = jnp.ones_like(c)
    zeros = jnp.zeros_like(c)
    cfull = jnp.concatenate([c * SCALE] * 8 + [c] * 8 + [ones] * 8, axis=1)
    sfull = jnp.concatenate([sg * SCALE] * 8 + [sg] * 8 + [zeros] * 8, axis=1)
    y_p32 = jnp.concatenate([y[:, 32:], y[:, :32]], axis=1)
    y_m32 = jnp.concatenate([y[:, -32:], y[:, :-32]], axis=1)
    hi_full = jnp.concatenate([hi] * 24, axis=1)
    partner = jnp.where(hi_full, y_m32, y_p32)
    o_ref[...] = (y * cfull + partner * sfull).astype(jnp.bfloat16)


def _mask_block(i0, j0, bq, bk, bar_q, inst_q, bar_k, inst_k):
    i = i0 + jax.lax.broadcasted_iota(jnp.int32, (bq, bk), 0)
    j = j0 + jax.lax.broadcasted_iota(jnp.int32, (bq, bk), 1)
    causal = j <= i
    same = (inst_q == inst_k) & (inst_q < 129) & causal
    off = bar_q - bar_k
    nearfar = ((off >= 0) & (off <= WINDOW)) | (off == FAR)
    cross = ((inst_q < 129) & (bar_q >= 0) & (inst_k != inst_q)
             & (inst_k < 129) & nearfar)
    glob = ((inst_k == 129) | (bar_k == -1)) & causal
    return same, cross, glob


def _attn_variant(kq, qi, hp, cend, barc_ref, barr_ref, instc_ref, instr_ref,
                  qkv_ref, wo_ref, bo_ref, o_ref, ms_ref, mc_ref, mg_ref,
                  acc2e_ref, s2e_ref, attn_ref):
    bar_q = barc_ref[...]
    inst_q = instc_ref[...]

    @pl.when(hp == 0)
    def _build_masks():
        same, cross, glob = _mask_block(
            qi * BQ, 0, BQ, kq, bar_q, inst_q,
            barr_ref[0:1, 0:kq], instr_ref[0:1, 0:kq])
        ms_ref[:, 0:kq] = same.astype(jnp.bfloat16)
        mc_ref[:, 0:kq] = cross.astype(jnp.bfloat16)
        mg_ref[:, 0:kq] = glob.astype(jnp.bfloat16)

    q2 = qkv_ref[pl.ds(qi * BQ, BQ), pl.ds(hp * 128, 128)]
    kf = qkv_ref[0:kq, pl.ds(EMBED + hp * 128, 128)]
    vf = qkv_ref[0:kq, pl.ds(2 * EMBED + hp * 128, 128)]
    ms = ms_ref[:, 0:kq]
    mc = mc_ref[:, 0:kq]
    mg = mg_ref[:, 0:kq]

    n_ext = (S - kq) // EXTB
    halves = []
    for t in range(2):
        q = q2[:, t * HEAD_DIM:(t + 1) * HEAD_DIM]
        k = kf[:, t * HEAD_DIM:(t + 1) * HEAD_DIM]
        v = vf[:, t * HEAD_DIM:(t + 1) * HEAD_DIM]
        scores = jax.lax.dot_general(
            q, k, (((1,), (1,)), ((), ())),
            preferred_element_type=jnp.float32)
        m = jnp.max(scores, axis=-1, keepdims=True)

        if n_ext > 0:
            acc2e_ref[...] = jnp.zeros_like(acc2e_ref)
            s2e_ref[...] = jnp.zeros_like(s2e_ref)
            for b in range(n_ext):
                j0 = kq + b * EXTB

                @pl.when(j0 < cend)
                def _ext(j0=j0):
                    ke = qkv_ref[j0:j0 + EXTB,
                                 pl.ds(EMBED + hp * 128, 128)][
                                     :, t * HEAD_DIM:(t + 1) * HEAD_DIM]
                    ve = qkv_ref[j0:j0 + EXTB,
                                 pl.ds(2 * EMBED + hp * 128, 128)][
                                     :, t * HEAD_DIM:(t + 1) * HEAD_DIM]
                    _, cr, _ = _mask_block(
                        qi * BQ, j0, BQ, EXTB, bar_q, inst_q,
                        barr_ref[0:1, j0:j0 + EXTB],
                        instr_ref[0:1, j0:j0 + EXTB])
                    sce = jax.lax.dot_general(
                        q, ke, (((1,), (1,)), ((), ())),
                        preferred_element_type=jnp.float32)
                    ee = jnp.exp(jnp.minimum(sce - m, 80.0))
                    ee = ee * cr.astype(jnp.float32)
                    s2e_ref[...] = s2e_ref[...] + jnp.sum(
                        ee, axis=1, keepdims=True)
                    acc2e_ref[...] = acc2e_ref[...] + jnp.dot(
                        ee.astype(jnp.bfloat16), ve,
                        preferred_element_type=jnp.float32)

        e16 = jnp.exp(scores - m).astype(jnp.bfloat16)
        out = None
        for ci, mask in enumerate((ms, mc, mg)):
            em = e16 * mask
            ssum = jnp.sum(em.astype(jnp.float32), axis=-1, keepdims=True)
            if ci == 1 and n_ext > 0:
                ssum = ssum + s2e_ref[...]
            acc = jnp.dot(em, v, preferred_element_type=jnp.float32)
            if ci == 1 and n_ext > 0:
                acc = acc + acc2e_ref[...]
            part = acc / jnp.where(ssum == 0.0, 1.0, ssum)
            out = part if out is None else out + part
        halves.append(out)
    attn_ref[:, pl.ds(hp * 128, 128)] = jnp.concatenate(
        halves, axis=1).astype(jnp.bfloat16)

    @pl.when(hp == HEADS // 2 - 1)
    def _project():
        o_ref[...] = jnp.dot(attn_ref[...], wo_ref[...],
                             preferred_element_type=jnp.float32) + bo_ref[...]


def _attn_kernel(cend_ref, barc_ref, barr_ref, instc_ref, instr_ref,
                 qkv_ref, wo_ref, bo_ref, o_ref, ms_ref, mc_ref, mg_ref,
                 acc2e_ref, s2e_ref, attn_ref):
    qi = pl.program_id(0)
    hp = pl.program_id(1)
    cend = cend_ref[qi]
    args = (barc_ref, barr_ref, instc_ref, instr_ref, qkv_ref, wo_ref,
            bo_ref, o_ref, ms_ref, mc_ref, mg_ref, acc2e_ref, s2e_ref,
            attn_ref)

    for g in range(8):
        @pl.when(qi == g)
        def _var(g=g):
            _attn_variant(256 * (g + 1), qi, hp, cend, *args)


def _out_proj_kernel(a_ref, w_ref, b_ref, o_ref):
    o_ref[...] = jnp.dot(a_ref[...], w_ref[...],
                         preferred_element_type=jnp.float32) + b_ref[...]


@jax.jit
def kernel(x, bar_ids, instrument_ids, Wq, bq, Wk, bk, Wv, bv, Wo, bo):
    B, s, e = x.shape
    x2 = x.reshape(s, e).astype(jnp.bfloat16)
    bqkv = jnp.concatenate([bq, bk, bv]).reshape(1, 3 * e)

    nq = s // BQ
    qkv = pl.pallas_call(
        _qkv_rope_kernel,
        grid=(nq,),
        in_specs=[
            pl.BlockSpec((BQ, e), lambda i: (i, 0)),
            pl.BlockSpec((e, e), lambda i: (0, 0)),
            pl.BlockSpec((e, e), lambda i: (0, 0)),
            pl.BlockSpec((e, e), lambda i: (0, 0)),
            pl.BlockSpec((1, 3 * e), lambda i: (0, 0)),
        ],
        out_specs=pl.BlockSpec((BQ, 3 * e), lambda i: (i, 0)),
        out_shape=jax.ShapeDtypeStruct((s, 3 * e), jnp.bfloat16),
    )(x2, Wq.T.astype(jnp.bfloat16), Wk.T.astype(jnp.bfloat16),
      Wv.T.astype(jnp.bfloat16), bqkv)

    bar = bar_ids.reshape(s)
    bar_c = bar_ids.reshape(s, 1)
    bar_r = bar_ids.reshape(1, s)
    inst_c = instrument_ids.reshape(s, 1)
    inst_r = instrument_ids.reshape(1, s)

    blk_last = bar[BQ - 1::BQ]
    cross_end = jnp.sum(bar[None, :] <= blk_last[:, None],
                        axis=1).astype(jnp.int32)

    out = pl.pallas_call(
        _attn_kernel,
        grid=(nq, HEADS // 2),
        in_specs=[
            pl.BlockSpec(memory_space=pltpu.SMEM),
            pl.BlockSpec((BQ, 1), lambda i, h: (i, 0)),
            pl.BlockSpec((1, S), lambda i, h: (0, 0)),
            pl.BlockSpec((BQ, 1), lambda i, h: (i, 0)),
            pl.BlockSpec((1, S), lambda i, h: (0, 0)),
            pl.BlockSpec((S, 3 * EMBED), lambda i, h: (0, 0)),
            pl.BlockSpec((EMBED, EMBED), lambda i, h: (0, 0)),
            pl.BlockSpec((1, EMBED), lambda i, h: (0, 0)),
        ],
        out_specs=pl.BlockSpec((BQ, EMBED), lambda i, h: (i, 0)),
        out_shape=jax.ShapeDtypeStruct((s, e), jnp.float32),
        scratch_shapes=[
            pltpu.VMEM((BQ, S), jnp.bfloat16),
            pltpu.VMEM((BQ, S), jnp.bfloat16),
            pltpu.VMEM((BQ, S), jnp.bfloat16),
            pltpu.VMEM((BQ, HEAD_DIM), jnp.float32),
            pltpu.VMEM((BQ, 1), jnp.float32),
            pltpu.VMEM((BQ, EMBED), jnp.bfloat16),
        ],
    )(cross_end, bar_c, bar_r, inst_c, inst_r, qkv,
      Wo.T.astype(jnp.bfloat16), bo.reshape(1, e))

    return out.reshape(B, s, e)


# speedup vs baseline: 10.6989x; 10.6989x over previous
import functools
import math

import jax
import jax.numpy as jnp
from jax.experimental import pallas as pl
from jax.experimental.pallas import tpu as pltpu

EMBED = 1024
HEADS = 16
HEAD_DIM = 64
SCALE = HEAD_DIM ** -0.5
WINDOW = 2
FAR = 4
S = 2048
BQ = 256
EXTB = 256

_LOG1E4 = math.log(10000.0)


def _qkv_rope_kernel(x_ref, wq_ref, wk_ref, wv_ref, b_ref, o_ref):
    qi = pl.program_id(0)
    y = jnp.concatenate([
        jnp.dot(x_ref[...], wq_ref[...], preferred_element_type=jnp.float32),
        jnp.dot(x_ref[...], wk_ref[...], preferred_element_type=jnp.float32),
        jnp.dot(x_ref[...], wv_ref[...], preferred_element_type=jnp.float32),
    ], axis=1)
    y = y + b_ref[...]
    bq, n = y.shape
    col = jax.lax.broadcasted_iota(jnp.int32, (bq, 128), 1)
    d = col % HEAD_DIM
    dr = d % (HEAD_DIM // 2)
    hi = d >= (HEAD_DIM // 2)
    inv = jnp.exp(dr.astype(jnp.float32) * (-_LOG1E4 / (HEAD_DIM // 2)))
    row = jax.lax.broadcasted_iota(jnp.int32, (bq, 128), 0)
    pos = (qi * bq + row).astype(jnp.float32)
    ang = pos * inv
    c = jnp.cos(ang)
    sg = jnp.sin(ang)
    sg = jnp.where(hi, sg, -sg)
    ones = jnp.ones_like(c)
    zeros = jnp.zeros_like(c)
    cfull = jnp.concatenate([c * SCALE] * 8 + [c] * 8 + [ones] * 8, axis=1)
    sfull = jnp.concatenate([sg * SCALE] * 8 + [sg] * 8 + [zeros] * 8, axis=1)
    y_p32 = jnp.concatenate([y[:, 32:], y[:, :32]], axis=1)
    y_m32 = jnp.concatenate([y[:, -32:], y[:, :-32]], axis=1)
    hi_full = jnp.concatenate([hi] * 24, axis=1)
    partner = jnp.where(hi_full, y_m32, y_p32)
    o_ref[...] = (y * cfull + partner * sfull).astype(jnp.bfloat16)


def _mask_block(i0, j0, bq, bk, bar_q, inst_q, bar_k, inst_k):
    i = i0 + jax.lax.broadcasted_iota(jnp.int32, (bq, bk), 0)
    j = j0 + jax.lax.broadcasted_iota(jnp.int32, (bq, bk), 1)
    causal = j <= i
    same = (inst_q == inst_k) & (inst_q < 129) & causal
    off = bar_q - bar_k
    nearfar = ((off >= 0) & (off <= WINDOW)) | (off == FAR)
    cross = ((inst_q < 129) & (bar_q >= 0) & (inst_k != inst_q)
             & (inst_k < 129) & nearfar)
    glob = ((inst_k == 129) | (bar_k == -1)) & causal
    return same, cross, glob


def _attn_variant(kq, qi, hp, cend, barc_ref, barr_ref, instc_ref, instr_ref,
                  qkv_ref, wo_ref, bo_ref, o_ref, ms_ref, mc_ref, mg_ref,
                  acc2e_ref, s2e_ref, attn_ref):
    bar_q = barc_ref[...]
    inst_q = instc_ref[...]

    @pl.when(hp == 0)
    def _build_masks():
        same, cross, glob = _mask_block(
            qi * BQ, 0, BQ, kq, bar_q, inst_q,
            barr_ref[0:1, 0:kq], instr_ref[0:1, 0:kq])
        ms_ref[:, 0:kq] = same.astype(jnp.bfloat16)
        mc_ref[:, 0:kq] = cross.astype(jnp.bfloat16)
        mg_ref[:, 0:kq] = glob.astype(jnp.bfloat16)

    q2 = qkv_ref[pl.ds(qi * BQ, BQ), pl.ds(hp * 128, 128)]
    kf = qkv_ref[0:kq, pl.ds(EMBED + hp * 128, 128)]
    vf = qkv_ref[0:kq, pl.ds(2 * EMBED + hp * 128, 128)]
    ms = ms_ref[:, 0:kq]
    mc = mc_ref[:, 0:kq]
    mg = mg_ref[:, 0:kq]

    n_ext = (S - kq) // EXTB
    halves = []
    for t in range(2):
        q = q2[:, t * HEAD_DIM:(t + 1) * HEAD_DIM]
        k = kf[:, t * HEAD_DIM:(t + 1) * HEAD_DIM]
        v = vf[:, t * HEAD_DIM:(t + 1) * HEAD_DIM]
        scores = jax.lax.dot_general(
            q, k, (((1,), (1,)), ((), ())),
            preferred_element_type=jnp.float32)
        m = jnp.max(scores, axis=-1, keepdims=True)

        if n_ext > 0:
            acc2e_ref[...] = jnp.zeros_like(acc2e_ref)
            s2e_ref[...] = jnp.zeros_like(s2e_ref)
            for b in range(n_ext):
                j0 = kq + b * EXTB

                @pl.when(j0 < cend)
                def _ext(j0=j0):
                    ke = qkv_ref[j0:j0 + EXTB,
                                 pl.ds(EMBED + hp * 128, 128)][
                                     :, t * HEAD_DIM:(t + 1) * HEAD_DIM]
                    ve = qkv_ref[j0:j0 + EXTB,
                                 pl.ds(2 * EMBED + hp * 128, 128)][
                                     :, t * HEAD_DIM:(t + 1) * HEAD_DIM]
                    _, cr, _ = _mask_block(
                        qi * BQ, j0, BQ, EXTB, bar_q, inst_q,
                        barr_ref[0:1, j0:j0 + EXTB],
                        instr_ref[0:1, j0:j0 + EXTB])
                    sce = jax.lax.dot_general(
                        q, ke, (((1,), (1,)), ((), ())),
                        preferred_element_type=jnp.float32)
                    ee = jnp.exp(jnp.minimum(sce - m, 80.0))
                    ee = ee * cr.astype(jnp.float32)
                    s2e_ref[...] = s2e_ref[...] + jnp.sum(
                        ee, axis=1, keepdims=True)
                    acc2e_ref[...] = acc2e_ref[...] + jnp.dot(
                        ee.astype(jnp.bfloat16), ve,
                        preferred_element_type=jnp.float32)

        e16 = jnp.exp(scores - m).astype(jnp.bfloat16)
        out = None
        for ci, mask in enumerate((ms, mc, mg)):
            em = e16 * mask
            ssum = jnp.sum(em.astype(jnp.float32), axis=-1, keepdims=True)
            if ci == 1 and n_ext > 0:
                ssum = ssum + s2e_ref[...]
            acc = jnp.dot(em, v, preferred_element_type=jnp.float32)
            if ci == 1 and n_ext > 0:
                acc = acc + acc2e_ref[...]
            part = acc / jnp.where(ssum == 0.0, 1.0, ssum)
            out = part if out is None else out + part
        halves.append(out)
    attn_ref[:, pl.ds(hp * 128, 128)] = jnp.concatenate(
        halves, axis=1).astype(jnp.bfloat16)

    @pl.when(hp == HEADS // 2 - 1)
    def _project():
        o_ref[...] = jnp.dot(attn_ref[...], wo_ref[...],
                             preferred_element_type=jnp.float32) + bo_ref[...]


def _attn_kernel(cend_ref, barc_ref, barr_ref, instc_ref, instr_ref,
                 qkv_ref, wo_ref, bo_ref, o_ref, ms_ref, mc_ref, mg_ref,
                 acc2e_ref, s2e_ref, attn_ref):
    qi = pl.program_id(0)
    hp = pl.program_id(1)
    cend = cend_ref[qi]
    args = (barc_ref, barr_ref, instc_ref, instr_ref, qkv_ref, wo_ref,
            bo_ref, o_ref, ms_ref, mc_ref, mg_ref, acc2e_ref, s2e_ref,
            attn_ref)

    for g in range(4):
        @pl.when((qi >= 2 * g) & (qi < 2 * g + 2))
        def _var(g=g):
            _attn_variant(512 * (g + 1), qi, hp, cend, *args)


def _out_proj_kernel(a_ref, w_ref, b_ref, o_ref):
    o_ref[...] = jnp.dot(a_ref[...], w_ref[...],
                         preferred_element_type=jnp.float32) + b_ref[...]


@jax.jit
def kernel(x, bar_ids, instrument_ids, Wq, bq, Wk, bk, Wv, bv, Wo, bo):
    B, s, e = x.shape
    x2 = x.reshape(s, e).astype(jnp.bfloat16)
    bqkv = jnp.concatenate([bq, bk, bv]).reshape(1, 3 * e)

    nq = s // BQ
    qkv = pl.pallas_call(
        _qkv_rope_kernel,
        grid=(nq,),
        in_specs=[
            pl.BlockSpec((BQ, e), lambda i: (i, 0)),
            pl.BlockSpec((e, e), lambda i: (0, 0)),
            pl.BlockSpec((e, e), lambda i: (0, 0)),
            pl.BlockSpec((e, e), lambda i: (0, 0)),
            pl.BlockSpec((1, 3 * e), lambda i: (0, 0)),
        ],
        out_specs=pl.BlockSpec((BQ, 3 * e), lambda i: (i, 0)),
        out_shape=jax.ShapeDtypeStruct((s, 3 * e), jnp.bfloat16),
    )(x2, Wq.T.astype(jnp.bfloat16), Wk.T.astype(jnp.bfloat16),
      Wv.T.astype(jnp.bfloat16), bqkv)

    bar = bar_ids.reshape(s)
    bar_c = bar_ids.reshape(s, 1)
    bar_r = bar_ids.reshape(1, s)
    inst_c = instrument_ids.reshape(s, 1)
    inst_r = instrument_ids.reshape(1, s)

    blk_last = bar[BQ - 1::BQ]
    cross_end = jnp.sum(bar[None, :] <= blk_last[:, None],
                        axis=1).astype(jnp.int32)

    out = pl.pallas_call(
        _attn_kernel,
        grid=(nq, HEADS // 2),
        in_specs=[
            pl.BlockSpec(memory_space=pltpu.SMEM),
            pl.BlockSpec((BQ, 1), lambda i, h: (i, 0)),
            pl.BlockSpec((1, S), lambda i, h: (0, 0)),
            pl.BlockSpec((BQ, 1), lambda i, h: (i, 0)),
            pl.BlockSpec((1, S), lambda i, h: (0, 0)),
            pl.BlockSpec((S, 3 * EMBED), lambda i, h: (0, 0)),
            pl.BlockSpec((EMBED, EMBED), lambda i, h: (0, 0)),
            pl.BlockSpec((1, EMBED), lambda i, h: (0, 0)),
        ],
        out_specs=pl.BlockSpec((BQ, EMBED), lambda i, h: (i, 0)),
        out_shape=jax.ShapeDtypeStruct((s, e), jnp.float32),
        scratch_shapes=[
            pltpu.VMEM((BQ, S), jnp.bfloat16),
            pltpu.VMEM((BQ, S), jnp.bfloat16),
            pltpu.VMEM((BQ, S), jnp.bfloat16),
            pltpu.VMEM((BQ, HEAD_DIM), jnp.float32),
            pltpu.VMEM((BQ, 1), jnp.float32),
            pltpu.VMEM((BQ, EMBED), jnp.bfloat16),
        ],
    )(cross_end, bar_c, bar_r, inst_c, inst_r, qkv,
      Wo.T.astype(jnp.bfloat16), bo.reshape(1, e))

    return out.reshape(B, s, e)


# speedup vs baseline: 11.2163x; 1.0484x over previous
import functools
import math

import jax
import jax.numpy as jnp
from jax.experimental import pallas as pl
from jax.experimental.pallas import tpu as pltpu

EMBED = 1024
HEADS = 16
HEAD_DIM = 64
SCALE = HEAD_DIM ** -0.5
WINDOW = 2
FAR = 4
S = 2048
BQ = 256
EXTB = 256

_LOG1E4 = math.log(10000.0)


def _qkv_rope_kernel(x_ref, wq_ref, wk_ref, wv_ref, b_ref, o_ref):
    qi = pl.program_id(0)
    y = jnp.concatenate([
        jnp.dot(x_ref[...], wq_ref[...], preferred_element_type=jnp.float32),
        jnp.dot(x_ref[...], wk_ref[...], preferred_element_type=jnp.float32),
        jnp.dot(x_ref[...], wv_ref[...], preferred_element_type=jnp.float32),
    ], axis=1)
    y = y + b_ref[...]
    bq, n = y.shape
    col = jax.lax.broadcasted_iota(jnp.int32, (bq, 128), 1)
    d = col % HEAD_DIM
    dr = d % (HEAD_DIM // 2)
    hi = d >= (HEAD_DIM // 2)
    inv = jnp.exp(dr.astype(jnp.float32) * (-_LOG1E4 / (HEAD_DIM // 2)))
    row = jax.lax.broadcasted_iota(jnp.int32, (bq, 128), 0)
    pos = (qi * bq + row).astype(jnp.float32)
    ang = pos * inv
    c = jnp.cos(ang)
    sg = jnp.sin(ang)
    sg = jnp.where(hi, sg, -sg)
    ones = jnp.ones_like(c)
    zeros = jnp.zeros_like(c)
    cfull = jnp.concatenate([c * SCALE] * 8 + [c] * 8 + [ones] * 8, axis=1)
    sfull = jnp.concatenate([sg * SCALE] * 8 + [sg] * 8 + [zeros] * 8, axis=1)
    y_p32 = jnp.concatenate([y[:, 32:], y[:, :32]], axis=1)
    y_m32 = jnp.concatenate([y[:, -32:], y[:, :-32]], axis=1)
    hi_full = jnp.concatenate([hi] * 24, axis=1)
    partner = jnp.where(hi_full, y_m32, y_p32)
    o_ref[...] = (y * cfull + partner * sfull).astype(jnp.bfloat16)


def _mask_block(i0, j0, bq, bk, bar_q, inst_q, bar_k, inst_k):
    i = i0 + jax.lax.broadcasted_iota(jnp.int32, (bq, bk), 0)
    j = j0 + jax.lax.broadcasted_iota(jnp.int32, (bq, bk), 1)
    causal = j <= i
    same = (inst_q == inst_k) & (inst_q < 129) & causal
    off = bar_q - bar_k
    nearfar = ((off >= 0) & (off <= WINDOW)) | (off == FAR)
    cross = ((inst_q < 129) & (bar_q >= 0) & (inst_k != inst_q)
             & (inst_k < 129) & nearfar)
    glob = ((inst_k == 129) | (bar_k == -1)) & causal
    return same, cross, glob


def _attn_variant(kq, qi, hp, cend, barc_ref, barr_ref, instc_ref, instr_ref,
                  qkv_ref, wo_ref, bo_ref, o_ref, ms_ref, mc_ref, mg_ref,
                  acc2e_ref, s2e_ref, attn_ref):
    bar_q = barc_ref[...]
    inst_q = instc_ref[...]

    @pl.when(hp == 0)
    def _build_masks():
        same, cross, glob = _mask_block(
            qi * BQ, 0, BQ, kq, bar_q, inst_q,
            barr_ref[0:1, 0:kq], instr_ref[0:1, 0:kq])
        ms_ref[:, 0:kq] = same.astype(jnp.bfloat16)
        mc_ref[:, 0:kq] = cross.astype(jnp.bfloat16)
        mg_ref[:, 0:kq] = glob.astype(jnp.bfloat16)

    q2 = qkv_ref[pl.ds(qi * BQ, BQ), pl.ds(hp * 128, 128)]
    kf = qkv_ref[0:kq, pl.ds(EMBED + hp * 128, 128)]
    vf = qkv_ref[0:kq, pl.ds(2 * EMBED + hp * 128, 128)]
    ms = ms_ref[:, 0:kq]
    mc = mc_ref[:, 0:kq]
    mg = mg_ref[:, 0:kq]

    n_ext = (S - kq) // EXTB
    halves = []
    for t in range(2):
        q = q2[:, t * HEAD_DIM:(t + 1) * HEAD_DIM]
        k = kf[:, t * HEAD_DIM:(t + 1) * HEAD_DIM]
        v = vf[:, t * HEAD_DIM:(t + 1) * HEAD_DIM]
        scores = jax.lax.dot_general(
            q, k, (((1,), (1,)), ((), ())),
            preferred_element_type=jnp.float32)
        m = jnp.max(scores, axis=-1, keepdims=True)

        if n_ext > 0:
            acc2e_ref[...] = jnp.zeros_like(acc2e_ref)
            s2e_ref[...] = jnp.zeros_like(s2e_ref)
            for b in range(n_ext):
                j0 = kq + b * EXTB

                @pl.when(j0 < cend)
                def _ext(j0=j0):
                    ke = qkv_ref[j0:j0 + EXTB,
                                 pl.ds(EMBED + hp * 128, 128)][
                                     :, t * HEAD_DIM:(t + 1) * HEAD_DIM]
                    ve = qkv_ref[j0:j0 + EXTB,
                                 pl.ds(2 * EMBED + hp * 128, 128)][
                                     :, t * HEAD_DIM:(t + 1) * HEAD_DIM]
                    _, cr, _ = _mask_block(
                        qi * BQ, j0, BQ, EXTB, bar_q, inst_q,
                        barr_ref[0:1, j0:j0 + EXTB],
                        instr_ref[0:1, j0:j0 + EXTB])
                    sce = jax.lax.dot_general(
                        q, ke, (((1,), (1,)), ((), ())),
                        preferred_element_type=jnp.float32)
                    ee = jnp.exp(jnp.minimum(sce - m, 80.0))
                    ee = ee * cr.astype(jnp.float32)
                    s2e_ref[...] = s2e_ref[...] + jnp.sum(
                        ee, axis=1, keepdims=True)
                    acc2e_ref[...] = acc2e_ref[...] + jnp.dot(
                        ee.astype(jnp.bfloat16), ve,
                        preferred_element_type=jnp.float32)

        e16 = jnp.exp(scores - m).astype(jnp.bfloat16)
        vext = jnp.concatenate(
            [v, jnp.ones((kq, HEAD_DIM), jnp.bfloat16)], axis=1)
        out = None
        for ci, mask in enumerate((ms, mc, mg)):
            em = e16 * mask
            acc128 = jnp.dot(em, vext, preferred_element_type=jnp.float32)
            acc = acc128[:, :HEAD_DIM]
            ssum = acc128[:, HEAD_DIM:HEAD_DIM + 1]
            if ci == 1 and n_ext > 0:
                ssum = ssum + s2e_ref[...]
                acc = acc + acc2e_ref[...]
            part = acc / jnp.where(ssum == 0.0, 1.0, ssum)
            out = part if out is None else out + part
        halves.append(out)
    attn_ref[:, pl.ds(hp * 128, 128)] = jnp.concatenate(
        halves, axis=1).astype(jnp.bfloat16)

    @pl.when(hp == HEADS // 2 - 1)
    def _project():
        o_ref[...] = jnp.dot(attn_ref[...], wo_ref[...],
                             preferred_element_type=jnp.float32) + bo_ref[...]


def _attn_kernel(cend_ref, barc_ref, barr_ref, instc_ref, instr_ref,
                 qkv_ref, wo_ref, bo_ref, o_ref, ms_ref, mc_ref, mg_ref,
                 acc2e_ref, s2e_ref, attn_ref):
    qi = pl.program_id(0)
    hp = pl.program_id(1)
    cend = cend_ref[qi]
    args = (barc_ref, barr_ref, instc_ref, instr_ref, qkv_ref, wo_ref,
            bo_ref, o_ref, ms_ref, mc_ref, mg_ref, acc2e_ref, s2e_ref,
            attn_ref)

    for g in range(4):
        @pl.when((qi >= 2 * g) & (qi < 2 * g + 2))
        def _var(g=g):
            _attn_variant(512 * (g + 1), qi, hp, cend, *args)


def _out_proj_kernel(a_ref, w_ref, b_ref, o_ref):
    o_ref[...] = jnp.dot(a_ref[...], w_ref[...],
                         preferred_element_type=jnp.float32) + b_ref[...]


@jax.jit
def kernel(x, bar_ids, instrument_ids, Wq, bq, Wk, bk, Wv, bv, Wo, bo):
    B, s, e = x.shape
    x2 = x.reshape(s, e).astype(jnp.bfloat16)
    bqkv = jnp.concatenate([bq, bk, bv]).reshape(1, 3 * e)

    nq = s // BQ
    qkv = pl.pallas_call(
        _qkv_rope_kernel,
        grid=(nq,),
        in_specs=[
            pl.BlockSpec((BQ, e), lambda i: (i, 0)),
            pl.BlockSpec((e, e), lambda i: (0, 0)),
            pl.BlockSpec((e, e), lambda i: (0, 0)),
            pl.BlockSpec((e, e), lambda i: (0, 0)),
            pl.BlockSpec((1, 3 * e), lambda i: (0, 0)),
        ],
        out_specs=pl.BlockSpec((BQ, 3 * e), lambda i: (i, 0)),
        out_shape=jax.ShapeDtypeStruct((s, 3 * e), jnp.bfloat16),
    )(x2, Wq.T.astype(jnp.bfloat16), Wk.T.astype(jnp.bfloat16),
      Wv.T.astype(jnp.bfloat16), bqkv)

    bar = bar_ids.reshape(s)
    bar_c = bar_ids.reshape(s, 1)
    bar_r = bar_ids.reshape(1, s)
    inst_c = instrument_ids.reshape(s, 1)
    inst_r = instrument_ids.reshape(1, s)

    blk_last = bar[BQ - 1::BQ]
    cross_end = jnp.sum(bar[None, :] <= blk_last[:, None],
                        axis=1).astype(jnp.int32)

    out = pl.pallas_call(
        _attn_kernel,
        grid=(nq, HEADS // 2),
        in_specs=[
            pl.BlockSpec(memory_space=pltpu.SMEM),
            pl.BlockSpec((BQ, 1), lambda i, h: (i, 0)),
            pl.BlockSpec((1, S), lambda i, h: (0, 0)),
            pl.BlockSpec((BQ, 1), lambda i, h: (i, 0)),
            pl.BlockSpec((1, S), lambda i, h: (0, 0)),
            pl.BlockSpec((S, 3 * EMBED), lambda i, h: (0, 0)),
            pl.BlockSpec((EMBED, EMBED), lambda i, h: (0, 0)),
            pl.BlockSpec((1, EMBED), lambda i, h: (0, 0)),
        ],
        out_specs=pl.BlockSpec((BQ, EMBED), lambda i, h: (i, 0)),
        out_shape=jax.ShapeDtypeStruct((s, e), jnp.float32),
        scratch_shapes=[
            pltpu.VMEM((BQ, S), jnp.bfloat16),
            pltpu.VMEM((BQ, S), jnp.bfloat16),
            pltpu.VMEM((BQ, S), jnp.bfloat16),
            pltpu.VMEM((BQ, HEAD_DIM), jnp.float32),
            pltpu.VMEM((BQ, 1), jnp.float32),
            pltpu.VMEM((BQ, EMBED), jnp.bfloat16),
        ],
    )(cross_end, bar_c, bar_r, inst_c, inst_r, qkv,
      Wo.T.astype(jnp.bfloat16), bo.reshape(1, e))

    return out.reshape(B, s, e)


# speedup vs baseline: 11.2265x; 1.0009x over previous
import functools
import math

import jax
import jax.numpy as jnp
from jax.experimental import pallas as pl
from jax.experimental.pallas import tpu as pltpu

EMBED = 1024
HEADS = 16
HEAD_DIM = 64
SCALE = HEAD_DIM ** -0.5
WINDOW = 2
FAR = 4
S = 2048
BQ = 256
EXTB = 256

_LOG1E4 = math.log(10000.0)


def _qkv_rope_kernel(x_ref, wq_ref, wk_ref, wv_ref, b_ref, o_ref):
    qi = pl.program_id(0)
    y = jnp.concatenate([
        jnp.dot(x_ref[...], wq_ref[...], preferred_element_type=jnp.float32),
        jnp.dot(x_ref[...], wk_ref[...], preferred_element_type=jnp.float32),
        jnp.dot(x_ref[...], wv_ref[...], preferred_element_type=jnp.float32),
    ], axis=1)
    y = y + b_ref[...]
    bq, n = y.shape
    col = jax.lax.broadcasted_iota(jnp.int32, (bq, 128), 1)
    d = col % HEAD_DIM
    dr = d % (HEAD_DIM // 2)
    hi = d >= (HEAD_DIM // 2)
    inv = jnp.exp(dr.astype(jnp.float32) * (-_LOG1E4 / (HEAD_DIM // 2)))
    row = jax.lax.broadcasted_iota(jnp.int32, (bq, 128), 0)
    pos = (qi * bq + row).astype(jnp.float32)
    ang = pos * inv
    c = jnp.cos(ang)
    sg = jnp.sin(ang)
    sg = jnp.where(hi, sg, -sg)
    ones = jnp.ones_like(c)
    zeros = jnp.zeros_like(c)
    cfull = jnp.concatenate([c * SCALE] * 8 + [c] * 8 + [ones] * 8, axis=1)
    sfull = jnp.concatenate([sg * SCALE] * 8 + [sg] * 8 + [zeros] * 8, axis=1)
    y_p32 = jnp.concatenate([y[:, 32:], y[:, :32]], axis=1)
    y_m32 = jnp.concatenate([y[:, -32:], y[:, :-32]], axis=1)
    hi_full = jnp.concatenate([hi] * 24, axis=1)
    partner = jnp.where(hi_full, y_m32, y_p32)
    o_ref[...] = (y * cfull + partner * sfull).astype(jnp.bfloat16)


def _mask_block(i0, j0, bq, bk, bar_q, inst_q, bar_k, inst_k):
    i = i0 + jax.lax.broadcasted_iota(jnp.int32, (bq, bk), 0)
    j = j0 + jax.lax.broadcasted_iota(jnp.int32, (bq, bk), 1)
    causal = j <= i
    eq = inst_q == inst_k
    q_ok = (inst_q < 129) & (bar_q >= 0)
    same = eq & (inst_q < 129) & causal
    off = bar_q - bar_k
    nearfar = (off.astype(jnp.uint32) <= WINDOW) | (off == FAR)
    cross = q_ok & ~eq & (inst_k < 129) & nearfar
    glob = ((inst_k == 129) | (bar_k == -1)) & causal
    return same, cross, glob


def _attn_variant(kq, qi, hp, cend, barc_ref, barr_ref, instc_ref, instr_ref,
                  qkv_ref, wo_ref, bo_ref, o_ref, ms_ref, mc_ref, mg_ref,
                  acc2e_ref, s2e_ref, attn_ref):
    bar_q = barc_ref[...]
    inst_q = instc_ref[...]

    @pl.when(hp == 0)
    def _build_masks():
        same, cross, glob = _mask_block(
            qi * BQ, 0, BQ, kq, bar_q, inst_q,
            barr_ref[0:1, 0:kq], instr_ref[0:1, 0:kq])
        ms_ref[:, 0:kq] = same.astype(jnp.bfloat16)
        mc_ref[:, 0:kq] = cross.astype(jnp.bfloat16)
        mg_ref[:, 0:kq] = glob.astype(jnp.bfloat16)

    q2 = qkv_ref[pl.ds(qi * BQ, BQ), pl.ds(hp * 128, 128)]
    kf = qkv_ref[0:kq, pl.ds(EMBED + hp * 128, 128)]
    vf = qkv_ref[0:kq, pl.ds(2 * EMBED + hp * 128, 128)]
    ms = ms_ref[:, 0:kq]
    mc = mc_ref[:, 0:kq]
    mg = mg_ref[:, 0:kq]

    n_ext = (S - kq) // EXTB
    halves = []
    for t in range(2):
        q = q2[:, t * HEAD_DIM:(t + 1) * HEAD_DIM]
        k = kf[:, t * HEAD_DIM:(t + 1) * HEAD_DIM]
        v = vf[:, t * HEAD_DIM:(t + 1) * HEAD_DIM]
        scores = jax.lax.dot_general(
            q, k, (((1,), (1,)), ((), ())),
            preferred_element_type=jnp.float32)
        m = jnp.max(scores, axis=-1, keepdims=True)

        if n_ext > 0:
            acc2e_ref[...] = jnp.zeros_like(acc2e_ref)
            s2e_ref[...] = jnp.zeros_like(s2e_ref)
            for b in range(n_ext):
                j0 = kq + b * EXTB

                @pl.when(j0 < cend)
                def _ext(j0=j0):
                    ke = qkv_ref[j0:j0 + EXTB,
                                 pl.ds(EMBED + hp * 128, 128)][
                                     :, t * HEAD_DIM:(t + 1) * HEAD_DIM]
                    ve = qkv_ref[j0:j0 + EXTB,
                                 pl.ds(2 * EMBED + hp * 128, 128)][
                                     :, t * HEAD_DIM:(t + 1) * HEAD_DIM]
                    _, cr, _ = _mask_block(
                        qi * BQ, j0, BQ, EXTB, bar_q, inst_q,
                        barr_ref[0:1, j0:j0 + EXTB],
                        instr_ref[0:1, j0:j0 + EXTB])
                    sce = jax.lax.dot_general(
                        q, ke, (((1,), (1,)), ((), ())),
                        preferred_element_type=jnp.float32)
                    ee = jnp.exp(jnp.minimum(sce - m, 80.0))
                    ee = ee * cr.astype(jnp.float32)
                    s2e_ref[...] = s2e_ref[...] + jnp.sum(
                        ee, axis=1, keepdims=True)
                    acc2e_ref[...] = acc2e_ref[...] + jnp.dot(
                        ee.astype(jnp.bfloat16), ve,
                        preferred_element_type=jnp.float32)

        e16 = jnp.exp(scores - m).astype(jnp.bfloat16)
        vext = jnp.concatenate(
            [v, jnp.ones((kq, HEAD_DIM), jnp.bfloat16)], axis=1)
        out = None
        for ci, mask in enumerate((ms, mc, mg)):
            em = e16 * mask
            acc128 = jnp.dot(em, vext, preferred_element_type=jnp.float32)
            acc = acc128[:, :HEAD_DIM]
            ssum = acc128[:, HEAD_DIM:HEAD_DIM + 1]
            if ci == 1 and n_ext > 0:
                ssum = ssum + s2e_ref[...]
                acc = acc + acc2e_ref[...]
            part = acc / jnp.where(ssum == 0.0, 1.0, ssum)
            out = part if out is None else out + part
        halves.append(out)
    attn_ref[:, pl.ds(hp * 128, 128)] = jnp.concatenate(
        halves, axis=1).astype(jnp.bfloat16)

    @pl.when(hp == HEADS // 2 - 1)
    def _project():
        o_ref[...] = jnp.dot(attn_ref[...], wo_ref[...],
                             preferred_element_type=jnp.float32) + bo_ref[...]


def _attn_kernel(cend_ref, barc_ref, barr_ref, instc_ref, instr_ref,
                 qkv_ref, wo_ref, bo_ref, o_ref, ms_ref, mc_ref, mg_ref,
                 acc2e_ref, s2e_ref, attn_ref):
    qi = pl.program_id(0)
    hp = pl.program_id(1)
    cend = cend_ref[qi]
    args = (barc_ref, barr_ref, instc_ref, instr_ref, qkv_ref, wo_ref,
            bo_ref, o_ref, ms_ref, mc_ref, mg_ref, acc2e_ref, s2e_ref,
            attn_ref)

    for g in range(4):
        @pl.when((qi >= 2 * g) & (qi < 2 * g + 2))
        def _var(g=g):
            _attn_variant(512 * (g + 1), qi, hp, cend, *args)


def _out_proj_kernel(a_ref, w_ref, b_ref, o_ref):
    o_ref[...] = jnp.dot(a_ref[...], w_ref[...],
                         preferred_element_type=jnp.float32) + b_ref[...]


@jax.jit
def kernel(x, bar_ids, instrument_ids, Wq, bq, Wk, bk, Wv, bv, Wo, bo):
    B, s, e = x.shape
    x2 = x.reshape(s, e).astype(jnp.bfloat16)
    bqkv = jnp.concatenate([bq, bk, bv]).reshape(1, 3 * e)

    nq = s // BQ
    qkv = pl.pallas_call(
        _qkv_rope_kernel,
        grid=(nq,),
        in_specs=[
            pl.BlockSpec((BQ, e), lambda i: (i, 0)),
            pl.BlockSpec((e, e), lambda i: (0, 0)),
            pl.BlockSpec((e, e), lambda i: (0, 0)),
            pl.BlockSpec((e, e), lambda i: (0, 0)),
            pl.BlockSpec((1, 3 * e), lambda i: (0, 0)),
        ],
        out_specs=pl.BlockSpec((BQ, 3 * e), lambda i: (i, 0)),
        out_shape=jax.ShapeDtypeStruct((s, 3 * e), jnp.bfloat16),
    )(x2, Wq.T.astype(jnp.bfloat16), Wk.T.astype(jnp.bfloat16),
      Wv.T.astype(jnp.bfloat16), bqkv)

    bar = bar_ids.reshape(s)
    bar_c = bar_ids.reshape(s, 1)
    bar_r = bar_ids.reshape(1, s)
    inst_c = instrument_ids.reshape(s, 1)
    inst_r = instrument_ids.reshape(1, s)

    blk_last = bar[BQ - 1::BQ]
    cross_end = jnp.sum(bar[None, :] <= blk_last[:, None],
                        axis=1).astype(jnp.int32)

    out = pl.pallas_call(
        _attn_kernel,
        grid=(nq, HEADS // 2),
        in_specs=[
            pl.BlockSpec(memory_space=pltpu.SMEM),
            pl.BlockSpec((BQ, 1), lambda i, h: (i, 0)),
            pl.BlockSpec((1, S), lambda i, h: (0, 0)),
            pl.BlockSpec((BQ, 1), lambda i, h: (i, 0)),
            pl.BlockSpec((1, S), lambda i, h: (0, 0)),
            pl.BlockSpec((S, 3 * EMBED), lambda i, h: (0, 0)),
            pl.BlockSpec((EMBED, EMBED), lambda i, h: (0, 0)),
            pl.BlockSpec((1, EMBED), lambda i, h: (0, 0)),
        ],
        out_specs=pl.BlockSpec((BQ, EMBED), lambda i, h: (i, 0)),
        out_shape=jax.ShapeDtypeStruct((s, e), jnp.float32),
        scratch_shapes=[
            pltpu.VMEM((BQ, S), jnp.bfloat16),
            pltpu.VMEM((BQ, S), jnp.bfloat16),
            pltpu.VMEM((BQ, S), jnp.bfloat16),
            pltpu.VMEM((BQ, HEAD_DIM), jnp.float32),
            pltpu.VMEM((BQ, 1), jnp.float32),
            pltpu.VMEM((BQ, EMBED), jnp.bfloat16),
        ],
    )(cross_end, bar_c, bar_r, inst_c, inst_r, qkv,
      Wo.T.astype(jnp.bfloat16), bo.reshape(1, e))

    return out.reshape(B, s, e)
